# baseline (speedup 1.0000x reference)
import jax
import jax.numpy as jnp
from jax.experimental import pallas as pl
from jax.experimental.pallas import tpu as pltpu

B, T = 16, 20
CODE_NUM, MED_NUM = 4880, 350
CODE_SIZE, HIDDEN, ATT, OUT = 48, 150, 32, 4880
H3 = 3 * HIDDEN
V = B * T
VT = 160
KB = 488
NKB = CODE_NUM // KB
NVT = V // VT
NEG = -1e9


def _stats_body(c_ref, m_ref, cemb_ref, memb_ref, theta_ref, we_ref, bth_ref,
                xt_ref, g_ref, tw_ref, ew_ref, invsm_ref, act_ref):
    f32 = jnp.float32
    xt = jnp.dot(cemb_ref[...], theta_ref[...], preferred_element_type=f32)
    xt_ref[...] = xt
    et = jnp.dot(memb_ref[...], theta_ref[...], preferred_element_type=f32)
    ew_ref[...] = jnp.dot(et, we_ref[...], preferred_element_type=f32)
    c = c_ref[...]
    m = m_ref[...]
    nc = jnp.sum(c, axis=1, keepdims=True)
    nm = jnp.sum(m, axis=1, keepdims=True)
    sum_c = jnp.maximum(nc, 1.0)
    sum_m = jnp.maximum(nm, 1.0)
    s_node = jnp.dot(c, xt, preferred_element_type=f32)
    s_e = jnp.dot(m, et, preferred_element_type=f32)
    q = s_node / sum_c
    g_ref[...] = (nm * q + s_e) / sum_m + bth_ref[...]
    tw_ref[...] = jnp.dot(q, we_ref[...], preferred_element_type=f32)
    invsm_ref[...] = 1.0 / sum_m
    act_ref[...] = (nc > 0).astype(f32)


def _code_att_body(xtT_ref, c_ref, gT_ref, waT_ref, ua_ref, act_ref,
                   rep1T_ref, rep2T_ref, m_acc, z_acc, r1_acc, r2_acc):
    k = pl.program_id(1)
    f32 = jnp.float32

    @pl.when(k == 0)
    def _():
        m_acc[...] = jnp.full_like(m_acc[...], NEG)
        z_acc[...] = jnp.zeros_like(z_acc[...])
        r1_acc[...] = jnp.zeros_like(r1_acc[...])
        r2_acc[...] = jnp.zeros_like(r2_acc[...])

    xtT = xtT_ref[0]
    c = c_ref[0]
    gT = gT_ref[0]
    waT = waT_ref[...]
    ua = ua_ref[...]
    for v in range(VT):
        cv = c[v:v + 1, :]
        hm = jax.nn.relu(cv * (xtT + gT[:, v:v + 1]))
        tv = jnp.tanh(jnp.dot(waT, hm, preferred_element_type=f32))
        sv = jnp.dot(ua, tv, preferred_element_type=f32)
        sv = jnp.where(cv > 0, sv, NEG)
        bm = jnp.max(sv, axis=1, keepdims=True)
        mp = m_acc[8 * v:8 * v + 1, :]
        nm_ = jnp.maximum(mp, bm)
        sc_ = jnp.exp(mp - nm_)
        pv = jnp.exp(sv - nm_)
        z_acc[8 * v:8 * v + 1, :] = (z_acc[8 * v:8 * v + 1, :] * sc_
                                     + jnp.sum(pv, axis=1, keepdims=True))
        r1c = jnp.sum(hm * pv, axis=1, keepdims=True)
        r1_acc[v] = r1_acc[v] * sc_ + r1c
        r2_acc[v] = jnp.maximum(r2_acc[v],
                                jnp.max(hm, axis=1, keepdims=True))
        m_acc[8 * v:8 * v + 1, :] = nm_

    @pl.when(k == NKB - 1)
    def _():
        act = act_ref[0]
        for v in range(VT):
            z = z_acc[8 * v:8 * v + 1, :]
            a = act[:, v:v + 1]
            rep1T_ref[0, :, v:v + 1] = (r1_acc[v] / z) * a
            rep2T_ref[0, :, v:v + 1] = r2_acc[v]


def _rep3_body(mm_ref, tw_ref, ew_ref, be_ref, invsm_ref, rep3_ref):
    e2 = jax.nn.relu(tw_ref[...][:, None, :] + ew_ref[...][None, :, :]
                     + be_ref[...][None, :, :])
    w = mm_ref[...][:, :, None] * e2
    rep3_ref[...] = jnp.sum(w, axis=1) * invsm_ref[...]


def _visit_att_body(r1_ref, r2_ref, r3_ref, maskf_ref,
                    wq1_ref, wq2_ref, wq3_ref, bq_ref, uq_ref,
                    wc1_ref, wc2_ref, wc3_ref, bc_ref, out_ref):
    f32 = jnp.float32
    r1 = r1_ref[...]
    r2 = r2_ref[...]
    r3 = r3_ref[...]
    pj = (jnp.dot(r1, wq1_ref[...], preferred_element_type=f32)
          + jnp.dot(r2, wq2_ref[...], preferred_element_type=f32)
          + jnp.dot(r3, wq3_ref[...], preferred_element_type=f32)
          + bq_ref[...])
    proj = jnp.tanh(pj)
    vs = jnp.sum(proj * uq_ref[...], axis=1, keepdims=True)
    vs = jnp.where(maskf_ref[...] > 0, vs, NEG)
    lane_v = jax.lax.broadcasted_iota(jnp.int32, (V, T), 0)
    t_of_v = jax.lax.broadcasted_iota(jnp.int32, (V, T), 1)
    q20 = (lane_v % T == t_of_v).astype(f32)
    bidx = jax.lax.broadcasted_iota(jnp.int32, (B, V), 0)
    vidx = jax.lax.broadcasted_iota(jnp.int32, (B, V), 1)
    sel = (vidx // T == bidx).astype(f32)
    vs16 = jnp.dot(sel, vs * q20, preferred_element_type=f32)
    mx = jnp.max(vs16, axis=1, keepdims=True)
    p = jnp.exp(vs16 - mx)
    alpha = p / jnp.sum(p, axis=1, keepdims=True)
    t_row = jax.lax.broadcasted_iota(jnp.int32, (T, V), 0)
    v_col = jax.lax.broadcasted_iota(jnp.int32, (T, V), 1)
    pmat = (v_col % T == t_row).astype(f32)
    aexp = jnp.dot(alpha, pmat, preferred_element_type=f32) * sel
    p1 = jnp.dot(aexp, r1, preferred_element_type=f32)
    p2 = jnp.dot(aexp, r2, preferred_element_type=f32)
    p3 = jnp.dot(aexp, r3, preferred_element_type=f32)
    out_ref[...] = (jnp.dot(p1, wc1_ref[...], preferred_element_type=f32)
                    + jnp.dot(p2, wc2_ref[...], preferred_element_type=f32)
                    + jnp.dot(p3, wc3_ref[...], preferred_element_type=f32)
                    + bc_ref[...])


def kernel(code_x, divided, neighbors, lens, medicine_codes, c_emb, m_emb,
           theta, b_theta, We, be, Wa, ua, Wq, bq, uq, Wc, bc):
    f32 = jnp.float32
    c = code_x.reshape(V, CODE_NUM)
    m = medicine_codes.reshape(V, MED_NUM)
    maskf = (jnp.arange(T)[None, :] < lens[:, None]).astype(f32).reshape(V, 1)

    sd = jax.ShapeDtypeStruct
    xt, g, tw, ew, invsm, act = pl.pallas_call(
        _stats_body,
        out_shape=(sd((CODE_NUM, HIDDEN), f32), sd((V, HIDDEN), f32),
                   sd((V, HIDDEN), f32), sd((MED_NUM, HIDDEN), f32),
                   sd((V, 1), f32), sd((V, 1), f32)),
    )(c, m, c_emb, m_emb, theta, We, b_theta.reshape(1, HIDDEN))

    xtT3 = xt.T.reshape(HIDDEN, NKB, KB).transpose(1, 0, 2)
    gT3 = g.T.reshape(HIDDEN, NVT, VT).transpose(1, 0, 2)
    act3 = act.reshape(NVT, 1, VT)
    rep1T, rep2T = pl.pallas_call(
        _code_att_body,
        grid=(NVT, NKB),
        in_specs=[
            pl.BlockSpec((1, HIDDEN, KB), lambda v, k: (k, 0, 0)),
            pl.BlockSpec((1, VT, KB), lambda v, k: (k, v, 0)),
            pl.BlockSpec((1, HIDDEN, VT), lambda v, k: (v, 0, 0)),
            pl.BlockSpec((ATT, HIDDEN), lambda v, k: (0, 0)),
            pl.BlockSpec((1, ATT), lambda v, k: (0, 0)),
            pl.BlockSpec((1, 1, VT), lambda v, k: (v, 0, 0)),
        ],
        out_specs=(pl.BlockSpec((1, HIDDEN, VT), lambda v, k: (v, 0, 0)),
                   pl.BlockSpec((1, HIDDEN, VT), lambda v, k: (v, 0, 0))),
        out_shape=(sd((NVT, HIDDEN, VT), f32), sd((NVT, HIDDEN, VT), f32)),
        scratch_shapes=[pltpu.VMEM((8 * VT, 1), f32),
                        pltpu.VMEM((8 * VT, 1), f32),
                        pltpu.VMEM((VT, HIDDEN, 1), f32),
                        pltpu.VMEM((VT, HIDDEN, 1), f32)],
    )(xtT3, c.reshape(V, NKB, KB).transpose(1, 0, 2), gT3, Wa.T,
      ua.reshape(1, ATT), act3)
    rep1 = rep1T.transpose(0, 2, 1).reshape(V, HIDDEN)
    rep2 = rep2T.transpose(0, 2, 1).reshape(V, HIDDEN)

    RVT = 16
    rep3 = pl.pallas_call(
        _rep3_body,
        grid=(V // RVT,),
        in_specs=[
            pl.BlockSpec((RVT, MED_NUM), lambda i: (i, 0)),
            pl.BlockSpec((RVT, HIDDEN), lambda i: (i, 0)),
            pl.BlockSpec((MED_NUM, HIDDEN), lambda i: (0, 0)),
            pl.BlockSpec((1, HIDDEN), lambda i: (0, 0)),
            pl.BlockSpec((RVT, 1), lambda i: (i, 0)),
        ],
        out_specs=pl.BlockSpec((RVT, HIDDEN), lambda i: (i, 0)),
        out_shape=sd((V, HIDDEN), f32),
    )(m, tw, ew, be.reshape(1, HIDDEN), invsm)

    out = pl.pallas_call(
        _visit_att_body,
        out_shape=sd((B, OUT), f32),
    )(rep1, rep2, rep3, maskf,
      Wq[:HIDDEN], Wq[HIDDEN:2 * HIDDEN], Wq[2 * HIDDEN:],
      bq.reshape(1, ATT), uq.reshape(1, ATT),
      Wc[:HIDDEN], Wc[HIDDEN:2 * HIDDEN], Wc[2 * HIDDEN:],
      bc.reshape(1, OUT))
    return out


# speedup vs baseline: 1.0457x; 1.0457x over previous
import jax
import jax.numpy as jnp
from jax.experimental import pallas as pl
from jax.experimental.pallas import tpu as pltpu

B, T = 16, 20
CODE_NUM, MED_NUM = 4880, 350
CODE_SIZE, HIDDEN, ATT, OUT = 48, 150, 32, 4880
H3 = 3 * HIDDEN
V = B * T
VT = 64
KB = 488
NKB = CODE_NUM // KB
NVT = V // VT
NEG = -1e9


def _stats_body(c_ref, m_ref, cemb_ref, memb_ref, theta_ref, we_ref, bth_ref,
                xt_ref, g_ref, tw_ref, ew_ref, invsm_ref, act_ref):
    f32 = jnp.float32
    xt = jnp.dot(cemb_ref[...], theta_ref[...], preferred_element_type=f32)
    xt_ref[...] = xt
    et = jnp.dot(memb_ref[...], theta_ref[...], preferred_element_type=f32)
    ew_ref[...] = jnp.dot(et, we_ref[...], preferred_element_type=f32)
    c = c_ref[...]
    m = m_ref[...]
    nc = jnp.sum(c, axis=1, keepdims=True)
    nm = jnp.sum(m, axis=1, keepdims=True)
    sum_c = jnp.maximum(nc, 1.0)
    sum_m = jnp.maximum(nm, 1.0)
    s_node = jnp.dot(c, xt, preferred_element_type=f32)
    s_e = jnp.dot(m, et, preferred_element_type=f32)
    q = s_node / sum_c
    g_ref[...] = (nm * q + s_e) / sum_m + bth_ref[...]
    tw_ref[...] = jnp.dot(q, we_ref[...], preferred_element_type=f32)
    invsm_ref[...] = 1.0 / sum_m
    act_ref[...] = (nc > 0).astype(f32)


def _code_att_body(xtT_ref, c_ref, gT_ref, waT_ref, ua_ref, act_ref,
                   rep1T_ref, rep2T_ref, m_acc, z_acc, r1_acc, r2_acc):
    k = pl.program_id(1)
    f32 = jnp.float32

    @pl.when(k == 0)
    def _():
        m_acc[...] = jnp.full_like(m_acc[...], NEG)
        z_acc[...] = jnp.zeros_like(z_acc[...])
        r1_acc[...] = jnp.zeros_like(r1_acc[...])
        r2_acc[...] = jnp.zeros_like(r2_acc[...])

    xtT = xtT_ref[0]
    c = c_ref[0]
    gT = gT_ref[0]
    waT = waT_ref[...]
    ua = ua_ref[...]
    for v in range(VT):
        cv = c[v:v + 1, :]
        hm = jax.nn.relu(cv * (xtT + gT[:, v:v + 1]))
        tv = jnp.tanh(jnp.dot(waT, hm, preferred_element_type=f32))
        sv = jnp.dot(ua, tv, preferred_element_type=f32)
        sv = jnp.where(cv > 0, sv, NEG)
        bm = jnp.max(sv, axis=1, keepdims=True)
        mp = m_acc[8 * v:8 * v + 1, :]
        nm_ = jnp.maximum(mp, bm)
        sc_ = jnp.exp(mp - nm_)
        pv = jnp.exp(sv - nm_)
        z_acc[8 * v:8 * v + 1, :] = (z_acc[8 * v:8 * v + 1, :] * sc_
                                     + jnp.sum(pv, axis=1, keepdims=True))
        r1c = jax.lax.dot_general(
            hm, pv, (((1,), (1,)), ((), ())),
            preferred_element_type=f32)
        r1_acc[v] = r1_acc[v] * sc_ + r1c
        r2_acc[v] = jnp.maximum(r2_acc[v],
                                jnp.max(hm, axis=1, keepdims=True))
        m_acc[8 * v:8 * v + 1, :] = nm_

    @pl.when(k == NKB - 1)
    def _():
        act = act_ref[0]
        for v in range(VT):
            z = z_acc[8 * v:8 * v + 1, :]
            a = act[:, v:v + 1]
            rep1T_ref[0, :, v:v + 1] = (r1_acc[v] / z) * a
            rep2T_ref[0, :, v:v + 1] = r2_acc[v]


def _rep3_body(mm_ref, tw_ref, ew_ref, be_ref, invsm_ref, rep3_ref):
    e2 = jax.nn.relu(tw_ref[...][:, None, :] + ew_ref[...][None, :, :]
                     + be_ref[...][None, :, :])
    w = mm_ref[...][:, :, None] * e2
    rep3_ref[...] = jnp.sum(w, axis=1) * invsm_ref[...]


def _visit_att_body(r1_ref, r2_ref, r3_ref, maskf_ref,
                    wq1_ref, wq2_ref, wq3_ref, bq_ref, uq_ref,
                    wc1_ref, wc2_ref, wc3_ref, bc_ref, out_ref):
    f32 = jnp.float32
    r1 = r1_ref[...]
    r2 = r2_ref[...]
    r3 = r3_ref[...]
    pj = (jnp.dot(r1, wq1_ref[...], preferred_element_type=f32)
          + jnp.dot(r2, wq2_ref[...], preferred_element_type=f32)
          + jnp.dot(r3, wq3_ref[...], preferred_element_type=f32)
          + bq_ref[...])
    proj = jnp.tanh(pj)
    vs = jnp.sum(proj * uq_ref[...], axis=1, keepdims=True)
    vs = jnp.where(maskf_ref[...] > 0, vs, NEG)
    lane_v = jax.lax.broadcasted_iota(jnp.int32, (V, T), 0)
    t_of_v = jax.lax.broadcasted_iota(jnp.int32, (V, T), 1)
    q20 = (lane_v % T == t_of_v).astype(f32)
    bidx = jax.lax.broadcasted_iota(jnp.int32, (B, V), 0)
    vidx = jax.lax.broadcasted_iota(jnp.int32, (B, V), 1)
    sel = (vidx // T == bidx).astype(f32)
    vs16 = jnp.dot(sel, vs * q20, preferred_element_type=f32)
    mx = jnp.max(vs16, axis=1, keepdims=True)
    p = jnp.exp(vs16 - mx)
    alpha = p / jnp.sum(p, axis=1, keepdims=True)
    t_row = jax.lax.broadcasted_iota(jnp.int32, (T, V), 0)
    v_col = jax.lax.broadcasted_iota(jnp.int32, (T, V), 1)
    pmat = (v_col % T == t_row).astype(f32)
    aexp = jnp.dot(alpha, pmat, preferred_element_type=f32) * sel
    p1 = jnp.dot(aexp, r1, preferred_element_type=f32)
    p2 = jnp.dot(aexp, r2, preferred_element_type=f32)
    p3 = jnp.dot(aexp, r3, preferred_element_type=f32)
    out_ref[...] = (jnp.dot(p1, wc1_ref[...], preferred_element_type=f32)
                    + jnp.dot(p2, wc2_ref[...], preferred_element_type=f32)
                    + jnp.dot(p3, wc3_ref[...], preferred_element_type=f32)
                    + bc_ref[...])


def kernel(code_x, divided, neighbors, lens, medicine_codes, c_emb, m_emb,
           theta, b_theta, We, be, Wa, ua, Wq, bq, uq, Wc, bc):
    f32 = jnp.float32
    c = code_x.reshape(V, CODE_NUM)
    m = medicine_codes.reshape(V, MED_NUM)
    maskf = (jnp.arange(T)[None, :] < lens[:, None]).astype(f32).reshape(V, 1)

    sd = jax.ShapeDtypeStruct
    xt, g, tw, ew, invsm, act = pl.pallas_call(
        _stats_body,
        out_shape=(sd((CODE_NUM, HIDDEN), f32), sd((V, HIDDEN), f32),
                   sd((V, HIDDEN), f32), sd((MED_NUM, HIDDEN), f32),
                   sd((V, 1), f32), sd((V, 1), f32)),
    )(c, m, c_emb, m_emb, theta, We, b_theta.reshape(1, HIDDEN))

    xtT3 = xt.T.reshape(HIDDEN, NKB, KB).transpose(1, 0, 2)
    gT3 = g.T.reshape(HIDDEN, NVT, VT).transpose(1, 0, 2)
    act3 = act.reshape(NVT, 1, VT)
    rep1T, rep2T = pl.pallas_call(
        _code_att_body,
        grid=(NVT, NKB),
        in_specs=[
            pl.BlockSpec((1, HIDDEN, KB), lambda v, k: (k, 0, 0)),
            pl.BlockSpec((1, VT, KB), lambda v, k: (k, v, 0)),
            pl.BlockSpec((1, HIDDEN, VT), lambda v, k: (v, 0, 0)),
            pl.BlockSpec((ATT, HIDDEN), lambda v, k: (0, 0)),
            pl.BlockSpec((1, ATT), lambda v, k: (0, 0)),
            pl.BlockSpec((1, 1, VT), lambda v, k: (v, 0, 0)),
        ],
        out_specs=(pl.BlockSpec((1, HIDDEN, VT), lambda v, k: (v, 0, 0)),
                   pl.BlockSpec((1, HIDDEN, VT), lambda v, k: (v, 0, 0))),
        out_shape=(sd((NVT, HIDDEN, VT), f32), sd((NVT, HIDDEN, VT), f32)),
        scratch_shapes=[pltpu.VMEM((8 * VT, 1), f32),
                        pltpu.VMEM((8 * VT, 1), f32),
                        pltpu.VMEM((VT, HIDDEN, 1), f32),
                        pltpu.VMEM((VT, HIDDEN, 1), f32)],
    )(xtT3, c.reshape(V, NKB, KB).transpose(1, 0, 2), gT3, Wa.T,
      ua.reshape(1, ATT), act3)
    rep1 = rep1T.transpose(0, 2, 1).reshape(V, HIDDEN)
    rep2 = rep2T.transpose(0, 2, 1).reshape(V, HIDDEN)

    RVT = 16
    rep3 = pl.pallas_call(
        _rep3_body,
        grid=(V // RVT,),
        in_specs=[
            pl.BlockSpec((RVT, MED_NUM), lambda i: (i, 0)),
            pl.BlockSpec((RVT, HIDDEN), lambda i: (i, 0)),
            pl.BlockSpec((MED_NUM, HIDDEN), lambda i: (0, 0)),
            pl.BlockSpec((1, HIDDEN), lambda i: (0, 0)),
            pl.BlockSpec((RVT, 1), lambda i: (i, 0)),
        ],
        out_specs=pl.BlockSpec((RVT, HIDDEN), lambda i: (i, 0)),
        out_shape=sd((V, HIDDEN), f32),
    )(m, tw, ew, be.reshape(1, HIDDEN), invsm)

    out = pl.pallas_call(
        _visit_att_body,
        out_shape=sd((B, OUT), f32),
    )(rep1, rep2, rep3, maskf,
      Wq[:HIDDEN], Wq[HIDDEN:2 * HIDDEN], Wq[2 * HIDDEN:],
      bq.reshape(1, ATT), uq.reshape(1, ATT),
      Wc[:HIDDEN], Wc[HIDDEN:2 * HIDDEN], Wc[2 * HIDDEN:],
      bc.reshape(1, OUT))
    return out


# speedup vs baseline: 1.2076x; 1.1549x over previous
import jax
import jax.numpy as jnp
from jax.experimental import pallas as pl
from jax.experimental.pallas import tpu as pltpu

B, T = 16, 20
CODE_NUM, MED_NUM = 4880, 350
CODE_SIZE, HIDDEN, ATT, OUT = 48, 150, 32, 4880
H3 = 3 * HIDDEN
V = B * T
VT = 64
KB = 976
NKB = CODE_NUM // KB
NVT = V // VT
NEG = -1e9


def _stats_body(c_ref, m_ref, cemb_ref, memb_ref, theta_ref, we_ref, bth_ref,
                xt_ref, g_ref, tw_ref, ew_ref, invsm_ref, act_ref):
    f32 = jnp.float32
    xt = jnp.dot(cemb_ref[...], theta_ref[...], preferred_element_type=f32)
    xt_ref[...] = xt
    et = jnp.dot(memb_ref[...], theta_ref[...], preferred_element_type=f32)
    ew_ref[...] = jnp.dot(et, we_ref[...], preferred_element_type=f32)
    c = c_ref[...]
    m = m_ref[...]
    nc = jnp.sum(c, axis=1, keepdims=True)
    nm = jnp.sum(m, axis=1, keepdims=True)
    sum_c = jnp.maximum(nc, 1.0)
    sum_m = jnp.maximum(nm, 1.0)
    s_node = jnp.dot(c, xt, preferred_element_type=f32)
    s_e = jnp.dot(m, et, preferred_element_type=f32)
    q = s_node / sum_c
    g_ref[...] = (nm * q + s_e) / sum_m + bth_ref[...]
    tw_ref[...] = jnp.dot(q, we_ref[...], preferred_element_type=f32)
    invsm_ref[...] = 1.0 / sum_m
    act_ref[...] = (nc > 0).astype(f32)


def _code_att_body(xtT_ref, c_ref, gT_ref, waT_ref, ua_ref, act_ref,
                   rep1T_ref, rep2T_ref, m_acc, z_acc, r1_acc, r2_acc):
    k = pl.program_id(1)
    f32 = jnp.float32

    @pl.when(k == 0)
    def _():
        m_acc[...] = jnp.full_like(m_acc[...], NEG)
        z_acc[...] = jnp.zeros_like(z_acc[...])
        r1_acc[...] = jnp.zeros_like(r1_acc[...])
        r2_acc[...] = jnp.zeros_like(r2_acc[...])

    xtT = xtT_ref[0]
    c = c_ref[0]
    gT = gT_ref[0]
    waT = waT_ref[...]
    ua = ua_ref[...]
    for v in range(VT):
        cv = c[v:v + 1, :]
        hm = jax.nn.relu(cv * (xtT + gT[:, v:v + 1]))
        tv = jnp.tanh(jnp.dot(waT, hm, preferred_element_type=f32))
        sv = jnp.dot(ua, tv, preferred_element_type=f32)
        sv = jnp.where(cv > 0, sv, NEG)
        bm = jnp.max(sv, axis=1, keepdims=True)
        mp = m_acc[8 * v:8 * v + 1, :]
        nm_ = jnp.maximum(mp, bm)
        sc_ = jnp.exp(mp - nm_)
        pv = jnp.exp(sv - nm_)
        z_acc[8 * v:8 * v + 1, :] = (z_acc[8 * v:8 * v + 1, :] * sc_
                                     + jnp.sum(pv, axis=1, keepdims=True))
        r1c = jax.lax.dot_general(
            hm, pv, (((1,), (1,)), ((), ())),
            preferred_element_type=f32)
        r1_acc[v] = r1_acc[v] * sc_ + r1c
        r2_acc[v] = jnp.maximum(r2_acc[v],
                                jnp.max(hm, axis=1, keepdims=True))
        m_acc[8 * v:8 * v + 1, :] = nm_

    @pl.when(k == NKB - 1)
    def _():
        act = act_ref[0]
        for v in range(VT):
            z = z_acc[8 * v:8 * v + 1, :]
            a = act[:, v:v + 1]
            rep1T_ref[0, :, v:v + 1] = (r1_acc[v] / z) * a
            rep2T_ref[0, :, v:v + 1] = r2_acc[v]


def _rep3_body(mm_ref, tw_ref, ew_ref, be_ref, invsm_ref, rep3_ref):
    e2 = jax.nn.relu(tw_ref[...][:, None, :] + ew_ref[...][None, :, :]
                     + be_ref[...][None, :, :])
    w = mm_ref[...][:, :, None] * e2
    rep3_ref[...] = jnp.sum(w, axis=1) * invsm_ref[...]


def _visit_att_body(r1_ref, r2_ref, r3_ref, maskf_ref,
                    wq1_ref, wq2_ref, wq3_ref, bq_ref, uq_ref,
                    wc1_ref, wc2_ref, wc3_ref, bc_ref, out_ref):
    f32 = jnp.float32
    r1 = r1_ref[...]
    r2 = r2_ref[...]
    r3 = r3_ref[...]
    pj = (jnp.dot(r1, wq1_ref[...], preferred_element_type=f32)
          + jnp.dot(r2, wq2_ref[...], preferred_element_type=f32)
          + jnp.dot(r3, wq3_ref[...], preferred_element_type=f32)
          + bq_ref[...])
    proj = jnp.tanh(pj)
    vs = jnp.sum(proj * uq_ref[...], axis=1, keepdims=True)
    vs = jnp.where(maskf_ref[...] > 0, vs, NEG)
    lane_v = jax.lax.broadcasted_iota(jnp.int32, (V, T), 0)
    t_of_v = jax.lax.broadcasted_iota(jnp.int32, (V, T), 1)
    q20 = (lane_v % T == t_of_v).astype(f32)
    bidx = jax.lax.broadcasted_iota(jnp.int32, (B, V), 0)
    vidx = jax.lax.broadcasted_iota(jnp.int32, (B, V), 1)
    sel = (vidx // T == bidx).astype(f32)
    vs16 = jnp.dot(sel, vs * q20, preferred_element_type=f32)
    mx = jnp.max(vs16, axis=1, keepdims=True)
    p = jnp.exp(vs16 - mx)
    alpha = p / jnp.sum(p, axis=1, keepdims=True)
    t_row = jax.lax.broadcasted_iota(jnp.int32, (T, V), 0)
    v_col = jax.lax.broadcasted_iota(jnp.int32, (T, V), 1)
    pmat = (v_col % T == t_row).astype(f32)
    aexp = jnp.dot(alpha, pmat, preferred_element_type=f32) * sel
    p1 = jnp.dot(aexp, r1, preferred_element_type=f32)
    p2 = jnp.dot(aexp, r2, preferred_element_type=f32)
    p3 = jnp.dot(aexp, r3, preferred_element_type=f32)
    out_ref[...] = (jnp.dot(p1, wc1_ref[...], preferred_element_type=f32)
                    + jnp.dot(p2, wc2_ref[...], preferred_element_type=f32)
                    + jnp.dot(p3, wc3_ref[...], preferred_element_type=f32)
                    + bc_ref[...])


def kernel(code_x, divided, neighbors, lens, medicine_codes, c_emb, m_emb,
           theta, b_theta, We, be, Wa, ua, Wq, bq, uq, Wc, bc):
    f32 = jnp.float32
    c = code_x.reshape(V, CODE_NUM)
    m = medicine_codes.reshape(V, MED_NUM)
    maskf = (jnp.arange(T)[None, :] < lens[:, None]).astype(f32).reshape(V, 1)

    sd = jax.ShapeDtypeStruct
    xt, g, tw, ew, invsm, act = pl.pallas_call(
        _stats_body,
        out_shape=(sd((CODE_NUM, HIDDEN), f32), sd((V, HIDDEN), f32),
                   sd((V, HIDDEN), f32), sd((MED_NUM, HIDDEN), f32),
                   sd((V, 1), f32), sd((V, 1), f32)),
    )(c, m, c_emb, m_emb, theta, We, b_theta.reshape(1, HIDDEN))

    xtT3 = xt.T.reshape(HIDDEN, NKB, KB).transpose(1, 0, 2)
    gT3 = g.T.reshape(HIDDEN, NVT, VT).transpose(1, 0, 2)
    act3 = act.reshape(NVT, 1, VT)
    rep1T, rep2T = pl.pallas_call(
        _code_att_body,
        grid=(NVT, NKB),
        in_specs=[
            pl.BlockSpec((1, HIDDEN, KB), lambda v, k: (k, 0, 0)),
            pl.BlockSpec((1, VT, KB), lambda v, k: (k, v, 0)),
            pl.BlockSpec((1, HIDDEN, VT), lambda v, k: (v, 0, 0)),
            pl.BlockSpec((ATT, HIDDEN), lambda v, k: (0, 0)),
            pl.BlockSpec((1, ATT), lambda v, k: (0, 0)),
            pl.BlockSpec((1, 1, VT), lambda v, k: (v, 0, 0)),
        ],
        out_specs=(pl.BlockSpec((1, HIDDEN, VT), lambda v, k: (v, 0, 0)),
                   pl.BlockSpec((1, HIDDEN, VT), lambda v, k: (v, 0, 0))),
        out_shape=(sd((NVT, HIDDEN, VT), f32), sd((NVT, HIDDEN, VT), f32)),
        scratch_shapes=[pltpu.VMEM((8 * VT, 1), f32),
                        pltpu.VMEM((8 * VT, 1), f32),
                        pltpu.VMEM((VT, HIDDEN, 1), f32),
                        pltpu.VMEM((VT, HIDDEN, 1), f32)],
    )(xtT3, c.reshape(V, NKB, KB).transpose(1, 0, 2), gT3, Wa.T,
      ua.reshape(1, ATT), act3)
    rep1 = rep1T.transpose(0, 2, 1).reshape(V, HIDDEN)
    rep2 = rep2T.transpose(0, 2, 1).reshape(V, HIDDEN)

    RVT = 16
    rep3 = pl.pallas_call(
        _rep3_body,
        grid=(V // RVT,),
        in_specs=[
            pl.BlockSpec((RVT, MED_NUM), lambda i: (i, 0)),
            pl.BlockSpec((RVT, HIDDEN), lambda i: (i, 0)),
            pl.BlockSpec((MED_NUM, HIDDEN), lambda i: (0, 0)),
            pl.BlockSpec((1, HIDDEN), lambda i: (0, 0)),
            pl.BlockSpec((RVT, 1), lambda i: (i, 0)),
        ],
        out_specs=pl.BlockSpec((RVT, HIDDEN), lambda i: (i, 0)),
        out_shape=sd((V, HIDDEN), f32),
    )(m, tw, ew, be.reshape(1, HIDDEN), invsm)

    out = pl.pallas_call(
        _visit_att_body,
        out_shape=sd((B, OUT), f32),
    )(rep1, rep2, rep3, maskf,
      Wq[:HIDDEN], Wq[HIDDEN:2 * HIDDEN], Wq[2 * HIDDEN:],
      bq.reshape(1, ATT), uq.reshape(1, ATT),
      Wc[:HIDDEN], Wc[HIDDEN:2 * HIDDEN], Wc[2 * HIDDEN:],
      bc.reshape(1, OUT))
    return out
